# baseline (device time: 169863 ns/iter reference)
import jax
import jax.numpy as jnp
from jax import lax
from jax.experimental import pallas as pl
from jax.experimental.pallas import tpu as pltpu

N_DEV = 32
M = 1536
CHUNK = M // N_DEV


def kernel(A, B):
    k_per = A.shape[1]
    n = B.shape[1]

    def body(a_ref, b_ref, out_ref, partial_ref, comm_ref, send_sems, recv_sems):
        my = lax.axis_index("i")
        left = lax.rem(my + N_DEV - 1, N_DEV)
        right = lax.rem(my + 1, N_DEV)

        barrier_sem = pltpu.get_barrier_semaphore()
        for nbr in (left, right):
            pl.semaphore_signal(
                barrier_sem, inc=1,
                device_id=(nbr,), device_id_type=pl.DeviceIdType.MESH,
            )
        pl.semaphore_wait(barrier_sem, 2)

        partial_ref[:, :] = jnp.dot(
            a_ref[:, :], b_ref[:, :], preferred_element_type=jnp.float32
        )

        s0 = lax.rem(my + N_DEV - 1, N_DEV)
        comm_ref[N_DEV - 1, :, :] = partial_ref[pl.ds(s0 * CHUNK, CHUNK), :]

        for h in range(N_DEV - 1):
            src_slot = N_DEV - 1 if h == 0 else h - 1
            rdma = pltpu.make_async_remote_copy(
                src_ref=comm_ref.at[src_slot],
                dst_ref=comm_ref.at[h],
                send_sem=send_sems.at[h],
                recv_sem=recv_sems.at[h],
                device_id=(right,),
                device_id_type=pl.DeviceIdType.MESH,
            )
            rdma.start()
            rdma.wait()

            r = lax.rem(my + 2 * N_DEV - h - 2, N_DEV)
            if h < N_DEV - 2:
                comm_ref[h, :, :] = (
                    comm_ref[h, :, :] + partial_ref[pl.ds(r * CHUNK, CHUNK), :]
                )
            else:
                out_ref[:, :] = (
                    comm_ref[h, :, :] + partial_ref[pl.ds(my * CHUNK, CHUNK), :]
                )

    return pl.pallas_call(
        body,
        out_shape=jax.ShapeDtypeStruct((CHUNK, n), jnp.float32),
        in_specs=[
            pl.BlockSpec(memory_space=pltpu.VMEM),
            pl.BlockSpec(memory_space=pltpu.VMEM),
        ],
        out_specs=pl.BlockSpec(memory_space=pltpu.VMEM),
        scratch_shapes=[
            pltpu.VMEM((M, n), jnp.float32),
            pltpu.VMEM((N_DEV, CHUNK, n), jnp.float32),
            pltpu.SemaphoreType.DMA((N_DEV - 1,)),
            pltpu.SemaphoreType.DMA((N_DEV - 1,)),
        ],
        compiler_params=pltpu.CompilerParams(collective_id=0),
    )(A, B)


# device time: 126575 ns/iter; 1.3420x vs baseline; 1.3420x over previous
import jax
import jax.numpy as jnp
from jax import lax
from jax.experimental import pallas as pl
from jax.experimental.pallas import tpu as pltpu

N_DEV = 32
M = 1536
CHUNK = M // N_DEV
Q_Z = M // 4
Q_Y = Q_Z // 4


def kernel(A, B):
    n = B.shape[1]

    def body(
        a_ref, b_ref, out_ref,
        partial_ref, comm1, red1, comm2, red2, stage3, comm3,
        s1_send, s1_recv, s2_send, s2_recv, s3_send, s3_recv,
    ):
        i = lax.axis_index("i")
        zi = i // 8
        w = i % 8
        y = w // 2
        x = (w + y) % 2

        zr = ((zi + 1) % 4) * 8 + w
        zl = ((zi + 3) % 4) * 8 + w
        yn = (y + 1) % 4
        yr = zi * 8 + 2 * yn + (x + yn) % 2
        yp = (y + 3) % 4
        yl = zi * 8 + 2 * yp + (x + yp) % 2
        xp = zi * 8 + (w + 1 - 2 * (w % 2))

        barrier_sem = pltpu.get_barrier_semaphore()
        for nbr in (zl, zr, yl, yr, xp):
            pl.semaphore_signal(
                barrier_sem, inc=1,
                device_id=(nbr,), device_id_type=pl.DeviceIdType.MESH,
            )
        pl.semaphore_wait(barrier_sem, 5)

        partial_ref[:, :] = jnp.dot(
            a_ref[:, :], b_ref[:, :], preferred_element_type=jnp.float32
        )

        s0 = (zi + 3) % 4
        comm1[3, :, :] = partial_ref[pl.ds(s0 * Q_Z, Q_Z), :]
        for h in range(3):
            src_slot = 3 if h == 0 else h - 1
            rdma = pltpu.make_async_remote_copy(
                src_ref=comm1.at[src_slot],
                dst_ref=comm1.at[h],
                send_sem=s1_send.at[h],
                recv_sem=s1_recv.at[h],
                device_id=(zr,),
                device_id_type=pl.DeviceIdType.MESH,
            )
            rdma.start()
            rdma.wait()
            r = (zi + 6 - h) % 4
            if h < 2:
                comm1[h, :, :] = (
                    comm1[h, :, :] + partial_ref[pl.ds(r * Q_Z, Q_Z), :]
                )
            else:
                red1[:, :] = (
                    comm1[h, :, :] + partial_ref[pl.ds(zi * Q_Z, Q_Z), :]
                )

        s0y = (y + 3) % 4
        comm2[3, :, :] = red1[pl.ds(s0y * Q_Y, Q_Y), :]
        for h in range(3):
            src_slot = 3 if h == 0 else h - 1
            rdma = pltpu.make_async_remote_copy(
                src_ref=comm2.at[src_slot],
                dst_ref=comm2.at[h],
                send_sem=s2_send.at[h],
                recv_sem=s2_recv.at[h],
                device_id=(yr,),
                device_id_type=pl.DeviceIdType.MESH,
            )
            rdma.start()
            rdma.wait()
            r = (y + 6 - h) % 4
            if h < 2:
                comm2[h, :, :] = (
                    comm2[h, :, :] + red1[pl.ds(r * Q_Y, Q_Y), :]
                )
            else:
                red2[:, :] = comm2[h, :, :] + red1[pl.ds(y * Q_Y, Q_Y), :]

        me_half = w % 2
        stage3[:, :] = red2[pl.ds((1 - me_half) * CHUNK, CHUNK), :]
        rdma = pltpu.make_async_remote_copy(
            src_ref=stage3,
            dst_ref=comm3,
            send_sem=s3_send,
            recv_sem=s3_recv,
            device_id=(xp,),
            device_id_type=pl.DeviceIdType.MESH,
        )
        rdma.start()
        rdma.wait()
        out_ref[:, :] = comm3[:, :] + red2[pl.ds(me_half * CHUNK, CHUNK), :]

    return pl.pallas_call(
        body,
        out_shape=jax.ShapeDtypeStruct((CHUNK, n), jnp.float32),
        in_specs=[
            pl.BlockSpec(memory_space=pltpu.VMEM),
            pl.BlockSpec(memory_space=pltpu.VMEM),
        ],
        out_specs=pl.BlockSpec(memory_space=pltpu.VMEM),
        scratch_shapes=[
            pltpu.VMEM((M, n), jnp.float32),
            pltpu.VMEM((4, Q_Z, n), jnp.float32),
            pltpu.VMEM((Q_Z, n), jnp.float32),
            pltpu.VMEM((4, Q_Y, n), jnp.float32),
            pltpu.VMEM((Q_Y, n), jnp.float32),
            pltpu.VMEM((CHUNK, n), jnp.float32),
            pltpu.VMEM((CHUNK, n), jnp.float32),
            pltpu.SemaphoreType.DMA((3,)),
            pltpu.SemaphoreType.DMA((3,)),
            pltpu.SemaphoreType.DMA((3,)),
            pltpu.SemaphoreType.DMA((3,)),
            pltpu.SemaphoreType.DMA,
            pltpu.SemaphoreType.DMA,
        ],
        compiler_params=pltpu.CompilerParams(collective_id=0),
    )(A, B)


# device time: 126041 ns/iter; 1.3477x vs baseline; 1.0042x over previous
import jax
import jax.numpy as jnp
from jax import lax
from jax.experimental import pallas as pl
from jax.experimental.pallas import tpu as pltpu

N_DEV = 32
M = 1536
CHUNK = M // N_DEV
Q_Z = M // 4
Q_Y = Q_Z // 4


def kernel(A, B):
    n = B.shape[1]
    hn = n // 2

    def body(
        a_ref, b_ref, out_ref,
        partial_ref, c1f, c1b, red1, c2f, c2b, red2, stage3, comm3,
        s1f_send, s1f_recv, s1b_send, s1b_recv,
        s2f_send, s2f_recv, s2b_send, s2b_recv,
        s3_send, s3_recv,
    ):
        i = lax.axis_index("i")
        zi = i // 8
        w = i % 8
        y = w // 2
        x = (w + y) % 2

        zr = ((zi + 1) % 4) * 8 + w
        zl = ((zi + 3) % 4) * 8 + w
        yn = (y + 1) % 4
        yr = zi * 8 + 2 * yn + (x + yn) % 2
        yp = (y + 3) % 4
        yl = zi * 8 + 2 * yp + (x + yp) % 2
        xp = zi * 8 + (w + 1 - 2 * (w % 2))

        barrier_sem = pltpu.get_barrier_semaphore()
        for nbr in (zl, zr, yl, yr, xp):
            pl.semaphore_signal(
                barrier_sem, inc=1,
                device_id=(nbr,), device_id_type=pl.DeviceIdType.MESH,
            )
        pl.semaphore_wait(barrier_sem, 5)

        partial_ref[:, :] = jnp.dot(
            a_ref[:, :], b_ref[:, :], preferred_element_type=jnp.float32
        )

        def ring_step(h, fwd_to, bwd_to, cf, cb, sf_send, sf_recv,
                      sb_send, sb_recv):
            src_slot = 3 if h == 0 else h - 1
            rf = pltpu.make_async_remote_copy(
                src_ref=cf.at[src_slot], dst_ref=cf.at[h],
                send_sem=sf_send.at[h], recv_sem=sf_recv.at[h],
                device_id=(fwd_to,), device_id_type=pl.DeviceIdType.MESH,
            )
            rb = pltpu.make_async_remote_copy(
                src_ref=cb.at[src_slot], dst_ref=cb.at[h],
                send_sem=sb_send.at[h], recv_sem=sb_recv.at[h],
                device_id=(bwd_to,), device_id_type=pl.DeviceIdType.MESH,
            )
            rf.start()
            rb.start()
            rf.wait()
            rb.wait()

        sf0 = (zi + 3) % 4
        sb0 = (zi + 1) % 4
        c1f[3, :, :] = partial_ref[pl.ds(sf0 * Q_Z, Q_Z), 0:hn]
        c1b[3, :, :] = partial_ref[pl.ds(sb0 * Q_Z, Q_Z), hn:n]
        for h in range(3):
            ring_step(h, zr, zl, c1f, c1b,
                      s1f_send, s1f_recv, s1b_send, s1b_recv)
            rf = (zi + 6 - h) % 4
            rb = (zi + 2 + h) % 4
            if h < 2:
                c1f[h, :, :] = c1f[h, :, :] + partial_ref[pl.ds(rf * Q_Z, Q_Z), 0:hn]
                c1b[h, :, :] = c1b[h, :, :] + partial_ref[pl.ds(rb * Q_Z, Q_Z), hn:n]
            else:
                red1[:, 0:hn] = c1f[h, :, :] + partial_ref[pl.ds(zi * Q_Z, Q_Z), 0:hn]
                red1[:, hn:n] = c1b[h, :, :] + partial_ref[pl.ds(zi * Q_Z, Q_Z), hn:n]

        sf0y = (y + 3) % 4
        sb0y = (y + 1) % 4
        c2f[3, :, :] = red1[pl.ds(sf0y * Q_Y, Q_Y), 0:hn]
        c2b[3, :, :] = red1[pl.ds(sb0y * Q_Y, Q_Y), hn:n]
        for h in range(3):
            ring_step(h, yr, yl, c2f, c2b,
                      s2f_send, s2f_recv, s2b_send, s2b_recv)
            rf = (y + 6 - h) % 4
            rb = (y + 2 + h) % 4
            if h < 2:
                c2f[h, :, :] = c2f[h, :, :] + red1[pl.ds(rf * Q_Y, Q_Y), 0:hn]
                c2b[h, :, :] = c2b[h, :, :] + red1[pl.ds(rb * Q_Y, Q_Y), hn:n]
            else:
                red2[:, 0:hn] = c2f[h, :, :] + red1[pl.ds(y * Q_Y, Q_Y), 0:hn]
                red2[:, hn:n] = c2b[h, :, :] + red1[pl.ds(y * Q_Y, Q_Y), hn:n]

        me_half = w % 2
        stage3[:, :] = red2[pl.ds((1 - me_half) * CHUNK, CHUNK), :]
        rdma = pltpu.make_async_remote_copy(
            src_ref=stage3,
            dst_ref=comm3,
            send_sem=s3_send,
            recv_sem=s3_recv,
            device_id=(xp,),
            device_id_type=pl.DeviceIdType.MESH,
        )
        rdma.start()
        rdma.wait()
        out_ref[:, :] = comm3[:, :] + red2[pl.ds(me_half * CHUNK, CHUNK), :]

    return pl.pallas_call(
        body,
        out_shape=jax.ShapeDtypeStruct((CHUNK, n), jnp.float32),
        in_specs=[
            pl.BlockSpec(memory_space=pltpu.VMEM),
            pl.BlockSpec(memory_space=pltpu.VMEM),
        ],
        out_specs=pl.BlockSpec(memory_space=pltpu.VMEM),
        scratch_shapes=[
            pltpu.VMEM((M, n), jnp.float32),
            pltpu.VMEM((4, Q_Z, hn), jnp.float32),
            pltpu.VMEM((4, Q_Z, hn), jnp.float32),
            pltpu.VMEM((Q_Z, n), jnp.float32),
            pltpu.VMEM((4, Q_Y, hn), jnp.float32),
            pltpu.VMEM((4, Q_Y, hn), jnp.float32),
            pltpu.VMEM((Q_Y, n), jnp.float32),
            pltpu.VMEM((CHUNK, n), jnp.float32),
            pltpu.VMEM((CHUNK, n), jnp.float32),
            pltpu.SemaphoreType.DMA((3,)),
            pltpu.SemaphoreType.DMA((3,)),
            pltpu.SemaphoreType.DMA((3,)),
            pltpu.SemaphoreType.DMA((3,)),
            pltpu.SemaphoreType.DMA((3,)),
            pltpu.SemaphoreType.DMA((3,)),
            pltpu.SemaphoreType.DMA((3,)),
            pltpu.SemaphoreType.DMA((3,)),
            pltpu.SemaphoreType.DMA,
            pltpu.SemaphoreType.DMA,
        ],
        compiler_params=pltpu.CompilerParams(collective_id=0),
    )(A, B)


# device time: 123125 ns/iter; 1.3796x vs baseline; 1.0237x over previous
import jax
import jax.numpy as jnp
from jax import lax
from jax.experimental import pallas as pl
from jax.experimental.pallas import tpu as pltpu

N_DEV = 32
M = 1536
CHUNK = M // N_DEV
Q_Z = M // 4
Q_Y = Q_Z // 4


def kernel(A, B):
    n = B.shape[1]
    hn = n // 2

    def body(
        a_ref, b_ref, out_ref,
        partial_ref, c1f, c1b, red1, c2f, c2b, red2, stage3, comm3,
        s1f_send, s1f_recv, s1b_send, s1b_recv,
        s2f_send, s2f_recv, s2b_send, s2b_recv,
        s3_send, s3_recv,
    ):
        i = lax.axis_index("i")
        zi = i // 8
        w = i % 8
        y = w // 2
        x = (w + y) % 2

        zr = ((zi + 1) % 4) * 8 + w
        zl = ((zi + 3) % 4) * 8 + w
        yn = (y + 1) % 4
        yr = zi * 8 + 2 * yn + (x + yn) % 2
        yp = (y + 3) % 4
        yl = zi * 8 + 2 * yp + (x + yp) % 2
        xp = zi * 8 + (w + 1 - 2 * (w % 2))

        barrier_sem = pltpu.get_barrier_semaphore()
        for nbr in (zl, zr, yl, yr, xp):
            pl.semaphore_signal(
                barrier_sem, inc=1,
                device_id=(nbr,), device_id_type=pl.DeviceIdType.MESH,
            )
        pl.semaphore_wait(barrier_sem, 5)

        def mm_piece(q, c0):
            partial_ref[pl.ds(q * Q_Z, Q_Z), pl.ds(c0, hn)] = jnp.dot(
                a_ref[pl.ds(q * Q_Z, Q_Z), :],
                b_ref[:, pl.ds(c0, hn)],
                preferred_element_type=jnp.float32,
            )

        def ring_start(h, fwd_to, bwd_to, cf, cb, sf_send, sf_recv,
                       sb_send, sb_recv):
            src_slot = 3 if h == 0 else h - 1
            rf = pltpu.make_async_remote_copy(
                src_ref=cf.at[src_slot], dst_ref=cf.at[h],
                send_sem=sf_send.at[h], recv_sem=sf_recv.at[h],
                device_id=(fwd_to,), device_id_type=pl.DeviceIdType.MESH,
            )
            rb = pltpu.make_async_remote_copy(
                src_ref=cb.at[src_slot], dst_ref=cb.at[h],
                send_sem=sb_send.at[h], recv_sem=sb_recv.at[h],
                device_id=(bwd_to,), device_id_type=pl.DeviceIdType.MESH,
            )
            rf.start()
            rb.start()
            return rf, rb

        sf0 = (zi + 3) % 4
        sb0 = (zi + 1) % 4
        mm_piece(sf0, 0)
        mm_piece(sb0, hn)
        c1f[3, :, :] = partial_ref[pl.ds(sf0 * Q_Z, Q_Z), 0:hn]
        c1b[3, :, :] = partial_ref[pl.ds(sb0 * Q_Z, Q_Z), hn:n]
        for h in range(3):
            r0f, r0b = ring_start(h, zr, zl, c1f, c1b,
                                  s1f_send, s1f_recv, s1b_send, s1b_recv)
            if h == 0:
                mm_piece((zi + 2) % 4, 0)
                mm_piece((zi + 2) % 4, hn)
                mm_piece((zi + 1) % 4, 0)
                mm_piece((zi + 3) % 4, hn)
                mm_piece(zi, 0)
                mm_piece(zi, hn)
            r0f.wait()
            r0b.wait()
            rf = (zi + 6 - h) % 4
            rb = (zi + 2 + h) % 4
            if h < 2:
                c1f[h, :, :] = c1f[h, :, :] + partial_ref[pl.ds(rf * Q_Z, Q_Z), 0:hn]
                c1b[h, :, :] = c1b[h, :, :] + partial_ref[pl.ds(rb * Q_Z, Q_Z), hn:n]
            else:
                red1[:, 0:hn] = c1f[h, :, :] + partial_ref[pl.ds(zi * Q_Z, Q_Z), 0:hn]
                red1[:, hn:n] = c1b[h, :, :] + partial_ref[pl.ds(zi * Q_Z, Q_Z), hn:n]

        sf0y = (y + 3) % 4
        sb0y = (y + 1) % 4
        c2f[3, :, :] = red1[pl.ds(sf0y * Q_Y, Q_Y), 0:hn]
        c2b[3, :, :] = red1[pl.ds(sb0y * Q_Y, Q_Y), hn:n]
        for h in range(3):
            r0f, r0b = ring_start(h, yr, yl, c2f, c2b,
                                  s2f_send, s2f_recv, s2b_send, s2b_recv)
            r0f.wait()
            r0b.wait()
            rf = (y + 6 - h) % 4
            rb = (y + 2 + h) % 4
            if h < 2:
                c2f[h, :, :] = c2f[h, :, :] + red1[pl.ds(rf * Q_Y, Q_Y), 0:hn]
                c2b[h, :, :] = c2b[h, :, :] + red1[pl.ds(rb * Q_Y, Q_Y), hn:n]
            else:
                red2[:, 0:hn] = c2f[h, :, :] + red1[pl.ds(y * Q_Y, Q_Y), 0:hn]
                red2[:, hn:n] = c2b[h, :, :] + red1[pl.ds(y * Q_Y, Q_Y), hn:n]

        me_half = w % 2
        stage3[:, :] = red2[pl.ds((1 - me_half) * CHUNK, CHUNK), :]
        rdma = pltpu.make_async_remote_copy(
            src_ref=stage3,
            dst_ref=comm3,
            send_sem=s3_send,
            recv_sem=s3_recv,
            device_id=(xp,),
            device_id_type=pl.DeviceIdType.MESH,
        )
        rdma.start()
        rdma.wait()
        out_ref[:, :] = comm3[:, :] + red2[pl.ds(me_half * CHUNK, CHUNK), :]

    return pl.pallas_call(
        body,
        out_shape=jax.ShapeDtypeStruct((CHUNK, n), jnp.float32),
        in_specs=[
            pl.BlockSpec(memory_space=pltpu.VMEM),
            pl.BlockSpec(memory_space=pltpu.VMEM),
        ],
        out_specs=pl.BlockSpec(memory_space=pltpu.VMEM),
        scratch_shapes=[
            pltpu.VMEM((M, n), jnp.float32),
            pltpu.VMEM((4, Q_Z, hn), jnp.float32),
            pltpu.VMEM((4, Q_Z, hn), jnp.float32),
            pltpu.VMEM((Q_Z, n), jnp.float32),
            pltpu.VMEM((4, Q_Y, hn), jnp.float32),
            pltpu.VMEM((4, Q_Y, hn), jnp.float32),
            pltpu.VMEM((Q_Y, n), jnp.float32),
            pltpu.VMEM((CHUNK, n), jnp.float32),
            pltpu.VMEM((CHUNK, n), jnp.float32),
            pltpu.SemaphoreType.DMA((3,)),
            pltpu.SemaphoreType.DMA((3,)),
            pltpu.SemaphoreType.DMA((3,)),
            pltpu.SemaphoreType.DMA((3,)),
            pltpu.SemaphoreType.DMA((3,)),
            pltpu.SemaphoreType.DMA((3,)),
            pltpu.SemaphoreType.DMA((3,)),
            pltpu.SemaphoreType.DMA((3,)),
            pltpu.SemaphoreType.DMA,
            pltpu.SemaphoreType.DMA,
        ],
        compiler_params=pltpu.CompilerParams(collective_id=0),
    )(A, B)


# device time: 115563 ns/iter; 1.4699x vs baseline; 1.0654x over previous
import jax
import jax.numpy as jnp
from jax import lax
from jax.experimental import pallas as pl
from jax.experimental.pallas import tpu as pltpu

N_DEV = 32
M = 1536
CHUNK = M // N_DEV
Q_Z = M // 4
Q_Y = Q_Z // 4
S_Z = Q_Z // 2
S_Y = Q_Y // 2


def kernel(A, B):
    n = B.shape[1]
    hn = n // 2

    def body(
        a_ref, b_ref, out_ref,
        partial_ref, c1f, c1b, red1, c2f, c2b, red2, stage3, comm3,
        s1f_send, s1f_recv, s1b_send, s1b_recv,
        s2f_send, s2f_recv, s2b_send, s2b_recv,
        s3_send, s3_recv,
    ):
        i = lax.axis_index("i")
        zi = i // 8
        w = i % 8
        y = w // 2
        x = (w + y) % 2

        zr = ((zi + 1) % 4) * 8 + w
        zl = ((zi + 3) % 4) * 8 + w
        yn = (y + 1) % 4
        yr = zi * 8 + 2 * yn + (x + yn) % 2
        yp = (y + 3) % 4
        yl = zi * 8 + 2 * yp + (x + yp) % 2
        xp = zi * 8 + (w + 1 - 2 * (w % 2))

        barrier_sem = pltpu.get_barrier_semaphore()
        for nbr in (zl, zr, yl, yr, xp):
            pl.semaphore_signal(
                barrier_sem, inc=1,
                device_id=(nbr,), device_id_type=pl.DeviceIdType.MESH,
            )
        pl.semaphore_wait(barrier_sem, 5)

        def mm_piece(q, c0):
            partial_ref[pl.ds(q * Q_Z, Q_Z), pl.ds(c0, hn)] = jnp.dot(
                a_ref[pl.ds(q * Q_Z, Q_Z), :],
                b_ref[:, pl.ds(c0, hn)],
                preferred_element_type=jnp.float32,
            )

        def sub_rdma(cref, sub_rows, h, s, send_sems, recv_sems, to):
            src_slot = 3 if h == 0 else h - 1
            r = pltpu.make_async_remote_copy(
                src_ref=cref.at[src_slot, pl.ds(s * sub_rows, sub_rows)],
                dst_ref=cref.at[h, pl.ds(s * sub_rows, sub_rows)],
                send_sem=send_sems.at[h, s],
                recv_sem=recv_sems.at[h, s],
                device_id=(to,), device_id_type=pl.DeviceIdType.MESH,
            )
            r.start()
            return r

        def ring_phase(cf, cb, src, dst, blk, sub, fwd_to, bwd_to, pos,
                       sf_send, sf_recv, sb_send, sb_recv, first=False):
            cf[3, :, :] = src[pl.ds(((pos + 3) % 4) * blk, blk), 0:hn]
            cb[3, :, :] = src[pl.ds(((pos + 1) % 4) * blk, blk), hn:n]
            inflight = [
                [sub_rdma(cf, sub, 0, s, sf_send, sf_recv, fwd_to),
                 sub_rdma(cb, sub, 0, s, sb_send, sb_recv, bwd_to)]
                for s in range(2)
            ]
            if first:
                mm_piece((pos + 2) % 4, 0)
                mm_piece((pos + 2) % 4, hn)
                mm_piece((pos + 1) % 4, 0)
                mm_piece((pos + 3) % 4, hn)
                mm_piece(pos, 0)
                mm_piece(pos, hn)
            for h in range(3):
                rf = (pos + 6 - h) % 4
                rb = (pos + 2 + h) % 4
                for s in range(2):
                    inflight[s][0].wait()
                    inflight[s][1].wait()
                    rows = pl.ds(s * sub, sub)
                    if h < 2:
                        cf[h, rows, :] = (
                            cf[h, rows, :]
                            + src[pl.ds(rf * blk + s * sub, sub), 0:hn]
                        )
                        cb[h, rows, :] = (
                            cb[h, rows, :]
                            + src[pl.ds(rb * blk + s * sub, sub), hn:n]
                        )
                        inflight[s] = [
                            sub_rdma(cf, sub, h + 1, s, sf_send, sf_recv, fwd_to),
                            sub_rdma(cb, sub, h + 1, s, sb_send, sb_recv, bwd_to),
                        ]
                    else:
                        dst[rows, 0:hn] = (
                            cf[h, rows, :]
                            + src[pl.ds(pos * blk + s * sub, sub), 0:hn]
                        )
                        dst[rows, hn:n] = (
                            cb[h, rows, :]
                            + src[pl.ds(pos * blk + s * sub, sub), hn:n]
                        )

        mm_piece((zi + 3) % 4, 0)
        mm_piece((zi + 1) % 4, hn)
        ring_phase(c1f, c1b, partial_ref, red1, Q_Z, S_Z, zr, zl, zi,
                   s1f_send, s1f_recv, s1b_send, s1b_recv, first=True)

        ring_phase(c2f, c2b, red1, red2, Q_Y, S_Y, yr, yl, y,
                   s2f_send, s2f_recv, s2b_send, s2b_recv)

        me_half = w % 2
        stage3[:, :] = red2[pl.ds((1 - me_half) * CHUNK, CHUNK), :]
        rdma = pltpu.make_async_remote_copy(
            src_ref=stage3,
            dst_ref=comm3,
            send_sem=s3_send,
            recv_sem=s3_recv,
            device_id=(xp,),
            device_id_type=pl.DeviceIdType.MESH,
        )
        rdma.start()
        rdma.wait()
        out_ref[:, :] = comm3[:, :] + red2[pl.ds(me_half * CHUNK, CHUNK), :]

    return pl.pallas_call(
        body,
        out_shape=jax.ShapeDtypeStruct((CHUNK, n), jnp.float32),
        in_specs=[
            pl.BlockSpec(memory_space=pltpu.VMEM),
            pl.BlockSpec(memory_space=pltpu.VMEM),
        ],
        out_specs=pl.BlockSpec(memory_space=pltpu.VMEM),
        scratch_shapes=[
            pltpu.VMEM((M, n), jnp.float32),
            pltpu.VMEM((4, Q_Z, hn), jnp.float32),
            pltpu.VMEM((4, Q_Z, hn), jnp.float32),
            pltpu.VMEM((Q_Z, n), jnp.float32),
            pltpu.VMEM((4, Q_Y, hn), jnp.float32),
            pltpu.VMEM((4, Q_Y, hn), jnp.float32),
            pltpu.VMEM((Q_Y, n), jnp.float32),
            pltpu.VMEM((CHUNK, n), jnp.float32),
            pltpu.VMEM((CHUNK, n), jnp.float32),
            pltpu.SemaphoreType.DMA((3, 2)),
            pltpu.SemaphoreType.DMA((3, 2)),
            pltpu.SemaphoreType.DMA((3, 2)),
            pltpu.SemaphoreType.DMA((3, 2)),
            pltpu.SemaphoreType.DMA((3, 2)),
            pltpu.SemaphoreType.DMA((3, 2)),
            pltpu.SemaphoreType.DMA((3, 2)),
            pltpu.SemaphoreType.DMA((3, 2)),
            pltpu.SemaphoreType.DMA,
            pltpu.SemaphoreType.DMA,
        ],
        compiler_params=pltpu.CompilerParams(collective_id=0),
    )(A, B)
